# baseline (speedup 1.0000x reference)
import functools

import jax
import jax.numpy as jnp
from jax import lax
from jax.experimental import pallas as pl
from jax.experimental.pallas import tpu as pltpu
from jax.experimental.pallas import tpu_sc as plsc

EMBED_DIM = 64
SCALE = 8.0

NC = 2
NS = 16
NW = NC * NS

B_TOTAL = 16384 * 50
B_PER_W = B_TOTAL // NW
CHUNK = 128
N_CHUNKS = B_PER_W // CHUNK

_mesh = plsc.VectorSubcoreMesh(core_axis_name="c", subcore_axis_name="s")


@functools.partial(
    pl.kernel,
    out_type=jax.ShapeDtypeStruct((B_TOTAL, EMBED_DIM), jnp.float32),
    mesh=_mesh,
    scratch_types=[
        pltpu.VMEM((N_CHUNKS, CHUNK), jnp.int32),
        pltpu.VMEM((CHUNK, EMBED_DIM), jnp.float32),
        pltpu.SemaphoreType.DMA,
    ],
    compiler_params=pltpu.CompilerParams(use_tc_tiling_on_sc=False),
)
def _embed_kernel(lut_hbm, idx_hbm, out_hbm, idx_v, rows_v, gsem):
    c = lax.axis_index("c")
    s = lax.axis_index("s")
    w = s * NC + c
    base = w * B_PER_W
    pltpu.sync_copy(idx_hbm.at[w], idx_v)

    def chunk_body(g, carry):
        pltpu.async_copy(lut_hbm.at[idx_v.at[g]], rows_v, gsem).wait()

        def row_body(r, carry2):
            for j in range(EMBED_DIM // 16):
                sl = pl.ds(16 * j, 16)
                rows_v[r, sl] = rows_v[r, sl] * SCALE
            return carry2

        lax.fori_loop(0, CHUNK, row_body, 0, unroll=4)
        pltpu.sync_copy(rows_v, out_hbm.at[pl.ds(base + g * CHUNK, CHUNK)])
        return carry

    lax.fori_loop(0, N_CHUNKS, chunk_body, 0)


def kernel(x, lut):
    idx = x.reshape(NW, N_CHUNKS, CHUNK).astype(jnp.int32)
    out = _embed_kernel(lut, idx)
    return out.reshape(x.shape[0], x.shape[1], EMBED_DIM)


# speedup vs baseline: 1.1589x; 1.1589x over previous
import functools

import jax
import jax.numpy as jnp
from jax import lax
from jax.experimental import pallas as pl
from jax.experimental.pallas import tpu as pltpu
from jax.experimental.pallas import tpu_sc as plsc

EMBED_DIM = 64
SCALE = 8.0

NC = 2
NS = 16
NW = NC * NS

B_TOTAL = 16384 * 50
B_PER_W = B_TOTAL // NW
HALF = 128
CHUNK = 2 * HALF
N_CHUNKS = B_PER_W // CHUNK
N_BUF = 4
N_IDX_ROWS = B_PER_W // HALF

_mesh = plsc.VectorSubcoreMesh(core_axis_name="c", subcore_axis_name="s")


@functools.partial(
    pl.kernel,
    out_type=jax.ShapeDtypeStruct((B_TOTAL, EMBED_DIM), jnp.float32),
    mesh=_mesh,
    scratch_types=[
        pltpu.VMEM((N_IDX_ROWS, HALF), jnp.int32),
        pltpu.VMEM((CHUNK, EMBED_DIM), jnp.float32),
        pltpu.VMEM((CHUNK, EMBED_DIM), jnp.float32),
        pltpu.VMEM((CHUNK, EMBED_DIM), jnp.float32),
        pltpu.VMEM((CHUNK, EMBED_DIM), jnp.float32),
        pltpu.SemaphoreType.DMA,
        pltpu.SemaphoreType.DMA,
    ],
    compiler_params=pltpu.CompilerParams(use_tc_tiling_on_sc=False),
)
def _embed_kernel(lut_hbm, idx_hbm, out_hbm, idx_v, b0, b1, b2, b3,
                  gsem, wsem):
    bufs = [b0, b1, b2, b3]
    c = lax.axis_index("c")
    s = lax.axis_index("s")
    w = s * NC + c
    base = w * B_PER_W
    pltpu.sync_copy(idx_hbm.at[w], idx_v)

    def start_gather(g, buf):
        pltpu.async_copy(lut_hbm.at[idx_v.at[2 * g]],
                         buf.at[pl.ds(0, HALF)], gsem)
        pltpu.async_copy(lut_hbm.at[idx_v.at[2 * g + 1]],
                         buf.at[pl.ds(HALF, HALF)], gsem)

    def wait_bytes(sem, buf):
        pltpu.make_async_copy(out_hbm.at[pl.ds(0, CHUNK)], buf, sem).wait()

    for b in range(N_BUF - 1):
        start_gather(b, bufs[b])

    def outer(i, carry):
        for b in range(N_BUF):
            g = i * N_BUF + b
            buf = bufs[b]
            wait_bytes(gsem, buf)

            def row_body(r, carry2):
                for j in range(EMBED_DIM // 16):
                    sl = pl.ds(16 * j, 16)
                    buf[r, sl] = buf[r, sl] * SCALE
                return carry2

            lax.fori_loop(0, CHUNK, row_body, 0, unroll=8)
            pltpu.async_copy(buf, out_hbm.at[pl.ds(base + g * CHUNK, CHUNK)],
                             wsem)

            nxt = g + (N_BUF - 1)
            nbuf = bufs[(b + N_BUF - 1) % N_BUF]

            @pl.when(jnp.logical_and(nxt >= N_BUF, nxt < N_CHUNKS))
            def _():
                wait_bytes(wsem, nbuf)

            @pl.when(nxt < N_CHUNKS)
            def _():
                start_gather(nxt, nbuf)
        return carry

    lax.fori_loop(0, N_CHUNKS // N_BUF, outer, 0)
    for b in range(N_BUF):
        wait_bytes(wsem, bufs[b])


def kernel(x, lut):
    idx = x.reshape(NW, N_IDX_ROWS, HALF).astype(jnp.int32)
    out = _embed_kernel(lut, idx)
    return out.reshape(x.shape[0], x.shape[1], EMBED_DIM)
